# baseline (speedup 1.0000x reference)
import functools

import jax
import jax.numpy as jnp
from jax import lax
from jax.experimental import pallas as pl
from jax.experimental.pallas import tpu as pltpu
from jax.experimental.pallas import tpu_sc as plsc

_BETA = 0.25
_B = 8192
_D = 1024
_K = 1024
_BLK = 256
_KT = 256
_NSTEP = _B // _BLK


def _dist_argmin(r, cb_ref, nsq_ref, d_s):
    rsq = jnp.sum(r * r, axis=1, keepdims=True)
    for kb in range(_K // _KT):
        sl = slice(kb * _KT, (kb + 1) * _KT)
        xc = lax.dot_general(
            r,
            cb_ref[sl, :],
            (((1,), (1,)), ((), ())),
            preferred_element_type=jnp.float32,
            precision=lax.Precision.DEFAULT,
        )
        d_s[:, sl] = (rsq - 2.0 * xc) + nsq_ref[0, sl]
    d = d_s[...]
    m = jnp.min(d, axis=1)
    jix = lax.broadcasted_iota(jnp.int32, (_BLK, _K), 1)
    idx = jnp.min(
        jnp.where(d == m[:, None], jix, jnp.int32(_K)), axis=1
    ).astype(jnp.int32)
    return idx, m


def _emit_idx(idx_ref, loss_ref, idx, m):
    idx_ref[...] = jnp.stack([idx] * 8, axis=0)
    loss_ref[...] = jnp.full((1, 1, 128), jnp.sum(m), jnp.float32)


def _lvl0_body(x_ref, cb_ref, nsq_ref, idx_ref, loss_ref, d_s):
    idx, m = _dist_argmin(x_ref[...], cb_ref, nsq_ref, d_s)
    _emit_idx(idx_ref, loss_ref, idx, m)


def _lvl1_body(x_ref, xq_ref, cb_ref, nsq_ref, r_ref, oa_ref, idx_ref,
               loss_ref, d_s):
    r = x_ref[...]
    xq = xq_ref[...]
    xqst = r + (xq - r)
    oa_ref[...] = xqst
    rn = r - xqst
    r_ref[...] = rn
    idx, m = _dist_argmin(rn, cb_ref, nsq_ref, d_s)
    _emit_idx(idx_ref, loss_ref, idx, m)


def _lvl2_body(r_in_ref, xq_ref, oa_in_ref, cb_ref, nsq_ref, r_ref, oa_ref,
               idx_ref, loss_ref, d_s):
    r = r_in_ref[...]
    xq = xq_ref[...]
    xqst = r + (xq - r)
    oa_ref[...] = oa_in_ref[...] + xqst
    rn = r - xqst
    r_ref[...] = rn
    idx, m = _dist_argmin(rn, cb_ref, nsq_ref, d_s)
    _emit_idx(idx_ref, loss_ref, idx, m)


def _final_body(r_ref, xq_ref, oa_ref, out_ref):
    r = r_ref[...]
    xq = xq_ref[...]
    out_ref[...] = oa_ref[...] + (r + (xq - r))


_row_spec = pl.BlockSpec((_BLK, _D), lambda i: (i, 0))
_cb_spec = pl.BlockSpec((_K, _D), lambda i: (0, 0))
_nsq_spec = pl.BlockSpec((1, _K), lambda i: (0, 0))
_idx_spec = pl.BlockSpec((8, _BLK), lambda i: (0, i))
_loss_spec = pl.BlockSpec((1, 1, 128), lambda i: (i, 0, 0))
_idx_shape = jax.ShapeDtypeStruct((8, _B), jnp.int32)
_loss_shape = jax.ShapeDtypeStruct((_NSTEP, 1, 128), jnp.float32)
_row_shape = jax.ShapeDtypeStruct((_B, _D), jnp.float32)
_params = pltpu.CompilerParams(dimension_semantics=("parallel",))
_scratch = [pltpu.VMEM((_BLK, _K), jnp.float32)]

_lvl0 = pl.pallas_call(
    _lvl0_body,
    grid=(_NSTEP,),
    in_specs=[_row_spec, _cb_spec, _nsq_spec],
    out_specs=[_idx_spec, _loss_spec],
    out_shape=[_idx_shape, _loss_shape],
    scratch_shapes=_scratch,
    compiler_params=_params,
)

_lvl1 = pl.pallas_call(
    _lvl1_body,
    grid=(_NSTEP,),
    in_specs=[_row_spec, _row_spec, _cb_spec, _nsq_spec],
    out_specs=[_row_spec, _row_spec, _idx_spec, _loss_spec],
    out_shape=[_row_shape, _row_shape, _idx_shape, _loss_shape],
    scratch_shapes=_scratch,
    compiler_params=_params,
)

_lvl2 = pl.pallas_call(
    _lvl2_body,
    grid=(_NSTEP,),
    in_specs=[_row_spec, _row_spec, _row_spec, _cb_spec, _nsq_spec],
    out_specs=[_row_spec, _row_spec, _idx_spec, _loss_spec],
    out_shape=[_row_shape, _row_shape, _idx_shape, _loss_shape],
    scratch_shapes=_scratch,
    compiler_params=_params,
)

_final = pl.pallas_call(
    _final_body,
    grid=(_NSTEP,),
    in_specs=[_row_spec, _row_spec, _row_spec],
    out_specs=_row_spec,
    out_shape=_row_shape,
    compiler_params=_params,
)


_NW = 32
_BPW = _B // _NW
_GCH = 64


def _sc_gather_body(table_hbm, idx_hbm, out_hbm, idx_v, rows_v, sem):
    wid = lax.axis_index("s") * 2 + lax.axis_index("c")
    base = wid * _BPW
    pltpu.sync_copy(idx_hbm.at[pl.ds(base, _BPW)], idx_v)
    for c in range(_BPW // _GCH):
        pltpu.async_copy(
            table_hbm.at[idx_v.at[pl.ds(c * _GCH, _GCH)]], rows_v, sem
        ).wait()
        pltpu.sync_copy(rows_v, out_hbm.at[pl.ds(base + c * _GCH, _GCH)])


_sc_gather = functools.partial(
    pl.kernel,
    mesh=plsc.VectorSubcoreMesh(core_axis_name="c", subcore_axis_name="s"),
    out_type=jax.ShapeDtypeStruct((_B, _D), jnp.float32),
    scratch_types=[
        pltpu.VMEM((_BPW,), jnp.int32),
        pltpu.VMEM((_GCH, _D), jnp.float32),
        pltpu.SemaphoreType.DMA,
    ],
)(_sc_gather_body)


def kernel(x, cb0, cb1, cb2):
    nsqs = [
        jnp.sum(cb**2, axis=1)[None, :] for cb in (cb0, cb1, cb2)
    ]

    idx0p, loss0 = _lvl0(x, cb0, nsqs[0])
    xq0 = _sc_gather(cb0, idx0p[0])
    r1, oa0, idx1p, loss1 = _lvl1(x, xq0, cb1, nsqs[1])
    xq1 = _sc_gather(cb1, idx1p[0])
    r2, oa1, idx2p, loss2 = _lvl2(r1, xq1, oa0, cb2, nsqs[2])
    xq2 = _sc_gather(cb2, idx2p[0])
    out = _final(r2, xq2, oa1)

    loss_sum = (
        jnp.sum(loss0[:, 0, 0])
        + jnp.sum(loss1[:, 0, 0])
        + jnp.sum(loss2[:, 0, 0])
    )
    rq_loss = loss_sum * ((1.0 + _BETA) / (3.0 * _B * _D))
    indices = jnp.stack([idx0p[0], idx1p[0], idx2p[0]], axis=1)
    return out, rq_loss, indices
